# Initial kernel scaffold; baseline (speedup 1.0000x reference)
import jax
import jax.numpy as jnp
from jax.experimental import pallas as pl


def kernel(x, edge_index, batch, W1, b1, W2, b2, Wm, bm, Wl, bl):
    raise NotImplementedError("write your pallas kernel here")


# speedup vs baseline: 10.7692x; 10.7692x over previous
import functools

import jax
import jax.numpy as jnp
from jax import lax
from jax.experimental import pallas as pl
from jax.experimental.pallas import tpu as pltpu
from jax.experimental.pallas import tpu_sc as plsc

N = 10000
NPAD = 10112
E = 320000
F = 128
B = 64
K = 16
NCLS = 10
NC = 2
NS = 16
NW = NC * NS
CW = 128
CH = 80
EPT = CH * CW
EPAD = NW * EPT
RPS = NPAD // NS
RB = 1000
GRID = N // RB

_DOT = dict(preferred_element_type=jnp.float32,
            precision=lax.Precision.HIGHEST)


def _mesh():
    return plsc.VectorSubcoreMesh(core_axis_name="c", subcore_axis_name="s",
                                  num_cores=NC, num_subcores=NS)


def _sc_degree(dstp):
    def body(dst_hbm, out_hbm, dst_v, hist_v):
        cid = lax.axis_index("c")
        sid = lax.axis_index("s")
        wid = cid * NS + sid
        pltpu.sync_copy(dst_hbm.at[wid], dst_v)
        zeros16 = jnp.zeros((16,), jnp.float32)

        def _zero(i, carry):
            hist_v[pl.ds(i * 16, 16)] = zeros16
            return carry

        lax.fori_loop(0, NPAD // 16, _zero, 0)
        ones16 = jnp.ones((16,), jnp.float32)

        def _row(r, carry):
            def _vec(c, carry2):
                idx = dst_v[r, pl.ds(c * 16, 16)]
                plsc.addupdate_scatter(hist_v, [idx], ones16)
                return carry2

            return lax.fori_loop(0, CW // 16, _vec, carry)

        lax.fori_loop(0, CH, _row, 0)
        pltpu.sync_copy(hist_v, out_hbm.at[wid])

    fn = pl.kernel(
        body,
        out_type=jax.ShapeDtypeStruct((NW, NPAD), jnp.float32),
        mesh=_mesh(),
        compiler_params=pltpu.CompilerParams(needs_layout_passes=False),
        scratch_types=[
            pltpu.VMEM((CH, CW), jnp.int32),
            pltpu.VMEM((NPAD,), jnp.float32),
        ],
    )
    return fn(dstp)


def _sc_agg(table, srcp, dstp, zt):
    def body(table_hbm, src_hbm, dst_hbm, zeros_hbm, out_hbm,
             src_v, dst_v, rows0, acc, sem0):
        cid = lax.axis_index("c")
        sid = lax.axis_index("s")
        wid = cid * NS + sid
        pltpu.sync_copy(zeros_hbm.at[pl.ds(sid * RPS, RPS)],
                        acc.at[pl.ds(sid * RPS, RPS)])
        pltpu.sync_copy(src_hbm.at[wid], src_v)
        pltpu.sync_copy(dst_hbm.at[wid], dst_v)
        plsc.subcore_barrier()

        def _chunk(j, carry):
            pltpu.async_copy(table_hbm.at[src_v.at[j]], rows0, sem0).wait()
            pltpu.sync_copy(rows0, acc.at[dst_v.at[j]], add=True)
            return carry

        lax.fori_loop(0, CH, _chunk, 0)
        plsc.subcore_barrier()
        pltpu.sync_copy(acc.at[pl.ds(sid * RPS, RPS)],
                        out_hbm.at[cid, pl.ds(sid * RPS, RPS)])

    fn = pl.kernel(
        body,
        out_type=jax.ShapeDtypeStruct((NC, NPAD, F), jnp.float32),
        mesh=_mesh(),
        compiler_params=pltpu.CompilerParams(needs_layout_passes=False),
        scratch_types=[
            pltpu.VMEM((CH, CW), jnp.int32),
            pltpu.VMEM((CH, CW), jnp.int32),
            pltpu.VMEM((CW, F), jnp.float32),
            pltpu.VMEM_SHARED((NPAD, F), jnp.float32),
            pltpu.SemaphoreType.DMA,
        ],
    )
    return fn(table, srcp, dstp, zt)


def _dis_from_hist(hist_blk):
    deg = jnp.sum(hist_blk, axis=1) + 1.0
    return lax.rsqrt(deg)


def _t1_body(x_ref, w1_ref, hist_ref, out_ref):
    dis = _dis_from_hist(hist_ref[...])
    g = lax.dot_general(x_ref[...], w1_ref[...], (((1,), (0,)), ((), ())),
                        **_DOT)
    out_ref[...] = g * dis[:, None]


def _t1(x, W1, hist):
    return pl.pallas_call(
        _t1_body,
        grid=(GRID,),
        in_specs=[
            pl.BlockSpec((RB, F), lambda i: (i, 0)),
            pl.BlockSpec((F, F), lambda i: (0, 0)),
            pl.BlockSpec((RB, NW), lambda i: (i, 0)),
        ],
        out_specs=pl.BlockSpec((RB, F), lambda i: (i, 0)),
        out_shape=jax.ShapeDtypeStruct((NPAD, F), jnp.float32),
    )(x, W1, hist)


def _t2_body(part_ref, tab_ref, hist_ref, b1_ref, w2_ref, out_ref):
    dis = _dis_from_hist(hist_ref[...])
    t = part_ref[0] + part_ref[1] + tab_ref[...]
    z = jnp.maximum(t * dis[:, None] + b1_ref[...], 0.0)
    g = lax.dot_general(z, w2_ref[...], (((1,), (0,)), ((), ())), **_DOT)
    out_ref[...] = g * dis[:, None]


def _t2(part, tab, hist, b1r, W2):
    return pl.pallas_call(
        _t2_body,
        grid=(GRID,),
        in_specs=[
            pl.BlockSpec((NC, RB, F), lambda i: (0, i, 0)),
            pl.BlockSpec((RB, F), lambda i: (i, 0)),
            pl.BlockSpec((RB, NW), lambda i: (i, 0)),
            pl.BlockSpec((1, F), lambda i: (0, 0)),
            pl.BlockSpec((F, F), lambda i: (0, 0)),
        ],
        out_specs=pl.BlockSpec((RB, F), lambda i: (i, 0)),
        out_shape=jax.ShapeDtypeStruct((NPAD, F), jnp.float32),
    )(part, tab, hist, b1r, W2)


def _t3_body(part_ref, tab_ref, hist_ref, b2_ref, wm_ref, bm_ref, batch_ref,
             wl_ref, bl_ref, out_ref, acc_ref):
    i = pl.program_id(0)
    dis = _dis_from_hist(hist_ref[...])
    t = part_ref[0] + part_ref[1] + tab_ref[...]
    h = jnp.maximum(t * dis[:, None] + b2_ref[...], 0.0)

    lm = lax.dot_general(h, wm_ref[...], (((1,), (0,)), ((), ())), **_DOT)
    lm = lm + bm_ref[...]
    m = jnp.max(lm, axis=-1, keepdims=True)
    ex = jnp.exp(lm - m)
    s = ex / jnp.sum(ex, axis=-1, keepdims=True)

    bids = batch_ref[...][0, 0]
    ggrp = lax.broadcasted_iota(jnp.int32, (RB, B * K), 1) // K
    s_tiled = jnp.concatenate([s] * B, axis=1)
    q = jnp.where(bids[:, None] == ggrp, s_tiled, 0.0)
    contrib = lax.dot_general(q, h, (((0,), (0,)), ((), ())), **_DOT)

    @pl.when(i == 0)
    def _():
        acc_ref[...] = contrib

    @pl.when(i > 0)
    def _():
        acc_ref[...] += contrib

    @pl.when(i == pl.num_programs(0) - 1)
    def _():
        a = acc_ref[...]
        scale = 1.0507009873554805
        alpha = 1.6732632423543772
        selu = scale * jnp.where(a > 0, a, alpha * (jnp.exp(a) - 1.0))
        bidx = lax.broadcasted_iota(jnp.int32, (B, B * K), 0)
        jgrp = lax.broadcasted_iota(jnp.int32, (B, B * K), 1) // K
        amat = jnp.where(bidx == jgrp, 1.0 / K, 0.0)
        pooled = lax.dot_general(amat, selu, (((1,), (0,)), ((), ())), **_DOT)
        logits = lax.dot_general(pooled, wl_ref[...],
                                 (((1,), (0,)), ((), ())), **_DOT)
        out_ref[...] = logits + bl_ref[...]


def _t3(part, tab, hist, b2r, Wm, bmr, batch3, Wl, blr):
    return pl.pallas_call(
        _t3_body,
        grid=(GRID,),
        in_specs=[
            pl.BlockSpec((NC, RB, F), lambda i: (0, i, 0)),
            pl.BlockSpec((RB, F), lambda i: (i, 0)),
            pl.BlockSpec((RB, NW), lambda i: (i, 0)),
            pl.BlockSpec((1, F), lambda i: (0, 0)),
            pl.BlockSpec((F, K), lambda i: (0, 0)),
            pl.BlockSpec((1, K), lambda i: (0, 0)),
            pl.BlockSpec((1, 1, RB), lambda i: (i, 0, 0)),
            pl.BlockSpec((F, NCLS), lambda i: (0, 0)),
            pl.BlockSpec((1, NCLS), lambda i: (0, 0)),
        ],
        out_specs=pl.BlockSpec((B, NCLS), lambda i: (0, 0)),
        out_shape=jax.ShapeDtypeStruct((B, NCLS), jnp.float32),
        scratch_shapes=[pltpu.VMEM((B * K, F), jnp.float32)],
    )(part, tab, hist, b2r, Wm, bmr, batch3, Wl, blr)


def kernel(x, edge_index, batch, W1, b1, W2, b2, Wm, bm, Wl, bl):
    x = x.astype(jnp.float32)
    src = edge_index[0].astype(jnp.int32)
    dst = edge_index[1].astype(jnp.int32)
    pad = EPAD - E
    srcp = jnp.concatenate([src, jnp.zeros((pad,), jnp.int32)]).reshape(
        NW, CH, CW)
    dstp = jnp.concatenate([dst, jnp.full((pad,), N, jnp.int32)]).reshape(
        NW, CH, CW)
    zt = jnp.zeros((NPAD, F), jnp.float32)
    batch3 = batch.astype(jnp.int32).reshape(GRID, 1, RB)
    b1r = b1.reshape(1, F)
    b2r = b2.reshape(1, F)
    bmr = bm.reshape(1, K)
    blr = bl.reshape(1, NCLS)

    hist = _sc_degree(dstp).T
    h1p = _t1(x, W1, hist)
    part1 = _sc_agg(h1p, srcp, dstp, zt)
    h2p = _t2(part1, h1p, hist, b1r, W2)
    part2 = _sc_agg(h2p, srcp, dstp, zt)
    return _t3(part2, h2p, hist, b2r, Wm, bmr, batch3, Wl, blr)
